# Initial kernel scaffold; baseline (speedup 1.0000x reference)
import jax
import jax.numpy as jnp
from jax.experimental import pallas as pl


def kernel(bin_centers, observation_probability_index, operator_number, lower_bound_1, upper_bound_1, lower_bound_2, upper_bound_2, lower_bound_3, upper_bound_3):
    raise NotImplementedError("write your pallas kernel here")


# speedup vs baseline: 10.3089x; 10.3089x over previous
import functools

import jax
import jax.numpy as jnp
from jax import lax
from jax.experimental import pallas as pl
from jax.experimental.pallas import tpu as pltpu
from jax.experimental.pallas import tpu_sc as plsc

_N_BINS = 512
_N_DIFFS = 16384
_NC, _NS, _L = 2, 16, 16
_NW = _NC * _NS
_ROWS = _N_DIFFS // _NW
_STEPS = _ROWS // _L


def _table_body(scal_ref, bc_ref, out_ref):
    bc = bc_ref[...]
    logit = jnp.log(bc / (1.0 - bc))
    rows = []
    for j in range(3):
        lo = scal_ref[2 * j]
        hi = scal_ref[2 * j + 1]
        lb = jnp.minimum(lo, hi)
        ub = jnp.maximum(lo, hi)
        x = (ub - logit) / (ub - lb + 0.0001)
        rows.append(1.0 / (1.0 + jnp.exp(-x)))
    out_ref[...] = jnp.concatenate(rows, axis=1)


_make_tables = pl.pallas_call(
    _table_body,
    out_shape=jax.ShapeDtypeStruct((1, 3 * _N_BINS), jnp.float32),
    in_specs=[
        pl.BlockSpec(memory_space=pltpu.SMEM),
        pl.BlockSpec(memory_space=pltpu.VMEM),
    ],
)


@functools.partial(
    pl.kernel,
    out_type=jax.ShapeDtypeStruct((_N_DIFFS,), jnp.float32),
    mesh=plsc.VectorSubcoreMesh(core_axis_name="c", subcore_axis_name="s"),
    compiler_params=pltpu.CompilerParams(needs_layout_passes=False),
    scratch_types=[
        pltpu.VMEM((3 * _N_BINS,), jnp.float32),
        pltpu.VMEM((3 * _ROWS,), jnp.int32),
        pltpu.VMEM((_ROWS,), jnp.float32),
    ],
)
def _sc_gather(tab_hbm, idx_hbm, out_hbm, tab_v, idx_v, out_v):
    wid = lax.axis_index("s") * _NC + lax.axis_index("c")
    base = wid * _ROWS
    pltpu.sync_copy(tab_hbm, tab_v)
    pltpu.sync_copy(idx_hbm.at[pl.ds(3 * base, 3 * _ROWS)], idx_v)
    lanes3 = lax.iota(jnp.int32, _L) * 3
    for i in range(_STEPS):
        p = None
        for j in range(3):
            bidx = plsc.load_gather(idx_v, [lanes3 + (i * 3 * _L + j)])
            pj = plsc.load_gather(tab_v, [bidx + (j * _N_BINS)])
            p = pj if p is None else p * pj
        out_v[pl.ds(i * _L, _L)] = p
    pltpu.sync_copy(out_v, out_hbm.at[pl.ds(base, _ROWS)])


def kernel(bin_centers, observation_probability_index, operator_number,
           lower_bound_1, upper_bound_1, lower_bound_2, upper_bound_2,
           lower_bound_3, upper_bound_3):
    del operator_number
    scalars = jnp.concatenate([
        lower_bound_1, upper_bound_1,
        lower_bound_2, upper_bound_2,
        lower_bound_3, upper_bound_3,
    ]).astype(jnp.float32)
    tab = _make_tables(scalars, bin_centers.reshape(1, _N_BINS))
    return _sc_gather(tab.reshape(-1), observation_probability_index.reshape(-1))


# speedup vs baseline: 10.5921x; 1.0275x over previous
import functools

import jax
import jax.numpy as jnp
from jax import lax
from jax.experimental import pallas as pl
from jax.experimental.pallas import tpu as pltpu
from jax.experimental.pallas import tpu_sc as plsc

_N_BINS = 512
_N_DIFFS = 16384
_NC, _NS, _L = 2, 16, 16
_NW = _NC * _NS
_ROWS = _N_DIFFS // _NW
_STEPS = _ROWS // _L


def _table_body(l1, u1, l2, u2, l3, u3, bc_ref, out_ref):
    bc = bc_ref[...]
    logit = jnp.log(bc / (1.0 - bc))
    rows = []
    for lo_ref, hi_ref in ((l1, u1), (l2, u2), (l3, u3)):
        lo = lo_ref[0]
        hi = hi_ref[0]
        lb = jnp.minimum(lo, hi)
        ub = jnp.maximum(lo, hi)
        x = (ub - logit) / (ub - lb + 0.0001)
        rows.append(1.0 / (1.0 + jnp.exp(-x)))
    out_ref[...] = jnp.concatenate(rows, axis=1)


_make_tables = pl.pallas_call(
    _table_body,
    out_shape=jax.ShapeDtypeStruct((1, 3 * _N_BINS), jnp.float32),
    in_specs=[pl.BlockSpec(memory_space=pltpu.SMEM)] * 6
    + [pl.BlockSpec(memory_space=pltpu.VMEM)],
)


@functools.partial(
    pl.kernel,
    out_type=jax.ShapeDtypeStruct((_N_DIFFS,), jnp.float32),
    mesh=plsc.VectorSubcoreMesh(core_axis_name="c", subcore_axis_name="s"),
    compiler_params=pltpu.CompilerParams(needs_layout_passes=False),
    scratch_types=[
        pltpu.VMEM((3 * _N_BINS,), jnp.float32),
        pltpu.VMEM((3 * _ROWS,), jnp.int32),
        pltpu.VMEM((_ROWS,), jnp.float32),
        pltpu.SemaphoreType.DMA,
        pltpu.SemaphoreType.DMA,
    ],
)
def _sc_gather(tab_hbm, idx_hbm, out_hbm, tab_v, idx_v, out_v, sem_t, sem_i):
    wid = lax.axis_index("s") * _NC + lax.axis_index("c")
    base = wid * _ROWS
    cp_t = pltpu.async_copy(tab_hbm, tab_v, sem_t)
    cp_i = pltpu.async_copy(idx_hbm.at[pl.ds(3 * base, 3 * _ROWS)], idx_v, sem_i)
    cp_t.wait()
    cp_i.wait()
    lanes3 = lax.iota(jnp.int32, _L) * 3
    for i in range(_STEPS):
        p = None
        for j in range(3):
            bidx = plsc.load_gather(idx_v, [lanes3 + (i * 3 * _L + j)])
            pj = plsc.load_gather(tab_v, [bidx + (j * _N_BINS)])
            p = pj if p is None else p * pj
        out_v[pl.ds(i * _L, _L)] = p
    pltpu.sync_copy(out_v, out_hbm.at[pl.ds(base, _ROWS)])


def kernel(bin_centers, observation_probability_index, operator_number,
           lower_bound_1, upper_bound_1, lower_bound_2, upper_bound_2,
           lower_bound_3, upper_bound_3):
    del operator_number
    tab = _make_tables(lower_bound_1, upper_bound_1, lower_bound_2,
                       upper_bound_2, lower_bound_3, upper_bound_3,
                       bin_centers.reshape(1, _N_BINS))
    return _sc_gather(tab.reshape(-1), observation_probability_index.reshape(-1))


# speedup vs baseline: 11.2939x; 1.0663x over previous
import functools

import jax
import jax.numpy as jnp
from jax import lax
from jax.experimental import pallas as pl
from jax.experimental.pallas import tpu as pltpu
from jax.experimental.pallas import tpu_sc as plsc

_N_BINS = 512
_N_DIFFS = 16384
_NC, _NS, _L = 2, 16, 16
_NW = _NC * _NS
_ROWS = _N_DIFFS // _NW
_STEPS = _ROWS // _L
_TSTEPS = _N_BINS // _L

_LN2 = 0.6931471805599453


def _ln(r):
    bits = plsc.bitcast(r, jnp.int32)
    e = (bits >> 23) - 127
    m = plsc.bitcast((bits & 0x007FFFFF) | 0x3F800000, jnp.float32)
    z = (m - 1.0) / (m + 1.0)
    z2 = z * z
    p = 1.0 / 9.0
    p = p * z2 + 1.0 / 7.0
    p = p * z2 + 1.0 / 5.0
    p = p * z2 + 1.0 / 3.0
    p = p * z2 + 1.0
    return 2.0 * z * p + e.astype(jnp.float32) * _LN2


@functools.partial(
    pl.kernel,
    out_type=jax.ShapeDtypeStruct((_N_DIFFS,), jnp.float32),
    mesh=plsc.VectorSubcoreMesh(core_axis_name="c", subcore_axis_name="s"),
    compiler_params=pltpu.CompilerParams(needs_layout_passes=False),
    scratch_types=[
        pltpu.VMEM((_N_BINS,), jnp.float32),
        pltpu.VMEM((48,), jnp.float32),
        pltpu.VMEM((3 * _N_BINS,), jnp.float32),
        pltpu.VMEM((3 * _ROWS,), jnp.int32),
        pltpu.VMEM((_ROWS,), jnp.float32),
        pltpu.SemaphoreType.DMA,
        pltpu.SemaphoreType.DMA,
    ],
)
def _sc_model(bc_hbm, l1, u1, l2, u2, l3, u3, idx_hbm, out_hbm,
              bc_v, scal_v, tab_v, idx_v, out_v, sem_s, sem_i):
    wid = lax.axis_index("s") * _NC + lax.axis_index("c")
    base = wid * _ROWS
    cp_i = pltpu.async_copy(idx_hbm.at[pl.ds(3 * base, 3 * _ROWS)], idx_v, sem_i)
    cp_b = pltpu.async_copy(bc_hbm, bc_v, sem_s)
    cps = [
        pltpu.async_copy(p_hbm, scal_v.at[pl.ds(8 * k, 1)], sem_s)
        for k, p_hbm in enumerate((l1, u1, l2, u2, l3, u3))
    ]
    cp_b.wait()
    for cp in cps:
        cp.wait()

    ubs, invs = [], []
    for j in range(3):
        vl = plsc.load_gather(scal_v, [jnp.full((_L,), 16 * j, jnp.int32)])
        vu = plsc.load_gather(scal_v, [jnp.full((_L,), 16 * j + 8, jnp.int32)])
        lo = jnp.minimum(vl, vu)
        hi = jnp.maximum(vl, vu)
        ubs.append(hi)
        invs.append(1.0 / (hi - lo + 0.0001))

    for k in range(_TSTEPS):
        t = bc_v[pl.ds(_L * k, _L)]
        logit = _ln(t / (1.0 - t))
        for j in range(3):
            x = (ubs[j] - logit) * invs[j]
            tab_v[pl.ds(j * _N_BINS + _L * k, _L)] = 1.0 / (1.0 + jnp.exp(-x))

    cp_i.wait()
    lanes3 = lax.iota(jnp.int32, _L) * 3
    for i in range(_STEPS):
        p = None
        for j in range(3):
            bidx = plsc.load_gather(idx_v, [lanes3 + (i * 3 * _L + j)])
            pj = plsc.load_gather(tab_v, [bidx + (j * _N_BINS)])
            p = pj if p is None else p * pj
        out_v[pl.ds(i * _L, _L)] = p
    pltpu.sync_copy(out_v, out_hbm.at[pl.ds(base, _ROWS)])


def kernel(bin_centers, observation_probability_index, operator_number,
           lower_bound_1, upper_bound_1, lower_bound_2, upper_bound_2,
           lower_bound_3, upper_bound_3):
    del operator_number
    return _sc_model(bin_centers, lower_bound_1, upper_bound_1,
                     lower_bound_2, upper_bound_2, lower_bound_3,
                     upper_bound_3, observation_probability_index.reshape(-1))
